# baseline (speedup 1.0000x reference)
import functools

import jax
import jax.numpy as jnp
from jax import lax
from jax.experimental import pallas as pl
from jax.experimental.pallas import tpu as pltpu
from jax.experimental.pallas import tpu_sc as plsc

N = 10000
NP = 10240
E = 320000
NC, NS = 2, 16
NW = NC * NS
B = 128
NB = 80
EW = NB * B
EPAD = NW * EW
RPS = NP // NS
CL = 64

_MESH = plsc.VectorSubcoreMesh(core_axis_name="c", subcore_axis_name="s")
_SC_PARAMS = pltpu.CompilerParams(use_tc_tiling_on_sc=False)


def _deg_body(row_hbm, deg_hbm, row_v, ones_v, zb, acc):
    cc = lax.axis_index("c")
    ss = lax.axis_index("s")
    wid = ss * NC + cc
    pltpu.sync_copy(row_hbm.at[wid], row_v)
    one = jnp.ones((16,), jnp.float32)
    zv = jnp.zeros((16,), jnp.float32)
    for r in range(B):
        ones_v[r, 0:16] = one
    for r in range(128):
        zb[r, 0:16] = zv

    def zstripe(t, carry):
        pltpu.sync_copy(zb, acc.at[pl.ds(ss * RPS + t * 128, 128)])
        return carry

    lax.fori_loop(0, RPS // 128, zstripe, 0)
    plsc.subcore_barrier()

    def batch(j, carry):
        pltpu.sync_copy(ones_v, acc.at[row_v.at[j]], add=True)
        return carry

    lax.fori_loop(0, NB, batch, 0)
    plsc.subcore_barrier()
    pltpu.sync_copy(acc.at[pl.ds(ss * RPS, RPS)],
                    deg_hbm.at[pl.ds(cc * NP + ss * RPS, RPS)])


_deg_kernel = pl.kernel(
    _deg_body,
    out_type=jax.ShapeDtypeStruct((NC * NP, 16), jnp.float32),
    mesh=_MESH,
    scratch_types=[
        pltpu.VMEM((NB, B), jnp.int32),
        pltpu.VMEM((B, 16), jnp.float32),
        pltpu.VMEM((128, 16), jnp.float32),
        pltpu.VMEM_SHARED((NP, 16), jnp.float32),
    ],
    compiler_params=_SC_PARAMS,
)


_RING = 8
_DG = 6
_ZR = 64
_GB = 32

_FAST = 0
_SB = 128
_NBF = 128
_NBS = 32
_TB = EPAD // _SB
_PB = _TB // NS
_PW = _GB * _SB


def _make_spmm_body(C):
    def body(g_hbm, colf_hbm, row2_hbm, p_hbm,
             col_v, row_v, gbufs, zb, acc, gsems, ssems, zsem, tsems):
        cc = lax.axis_index("c")
        ss = lax.axis_index("s")
        fast = cc == _FAST
        ng = jnp.where(fast, _NBF // _GB, _NBS // _GB)
        base = ss * _PB + jnp.where(fast, 0, _NBF)

        def stage_start(g):
            pc = g % 2
            pltpu.async_copy(colf_hbm.at[pl.ds((base + g * _GB) * _SB, _PW)],
                             col_v.at[pl.ds(pc * _PW, _PW)], tsems.at[pc])
            pltpu.async_copy(row2_hbm.at[pl.ds(base + g * _GB, _GB)],
                             row_v.at[pc], tsems.at[pc])

        def stage_wait(g):
            pc = g % 2
            pltpu.make_async_copy(
                colf_hbm.at[pl.ds((base + g * _GB) * _SB, _PW)],
                col_v.at[pl.ds(pc * _PW, _PW)], tsems.at[pc]).wait()
            pltpu.make_async_copy(row2_hbm.at[pl.ds(base + g * _GB, _GB)],
                                  row_v.at[pc], tsems.at[pc]).wait()

        def shift(pc, c):
            def go(t, carry):
                sl = pl.ds(pc * _PW + t * 16, 16)
                col_v[sl] = col_v[sl] + c * NP
                return carry
            lax.fori_loop(0, _PW // 16, go, 0)

        zv = jnp.zeros((16,), jnp.float32)
        for r in range(_ZR):
            for k in range(CL // 16):
                zb[r, k * 16:(k + 1) * 16] = zv

        def g_start(jl, i, pc):
            pltpu.async_copy(
                g_hbm.at[col_v.at[pl.ds(pc * _PW + jl * _SB, _SB)]],
                gbufs.at[i], gsems.at[i])

        def g_wait(jl, i, pc):
            pltpu.make_async_copy(
                g_hbm.at[col_v.at[pl.ds(pc * _PW + jl * _SB, _SB)]],
                gbufs.at[i], gsems.at[i]).wait()

        def s_start(jl, i, pc):
            pltpu.async_copy(gbufs.at[i], acc.at[row_v.at[pc].at[jl]],
                             ssems.at[i], add=True)

        def s_wait(jl, i, pc):
            pltpu.make_async_copy(gbufs.at[i], acc.at[row_v.at[pc].at[jl]],
                                  ssems.at[i]).wait()

        for c in range(C):
            for t in range(RPS // _ZR):
                pltpu.async_copy(zb, acc.at[pl.ds(ss * RPS + t * _ZR, _ZR)],
                                 zsem)
            for t in range(RPS // _ZR):
                pltpu.make_async_copy(
                    zb, acc.at[pl.ds(ss * RPS + t * _ZR, _ZR)], zsem).wait()
            plsc.subcore_barrier()

            stage_start(0)
            stage_wait(0)
            shift(0, c)

            @pl.when(ng > 1)
            def _():
                stage_start(1)

            def group(g, carry):
                pc = g % 2
                for i in range(_DG):
                    g_start(i, i, pc)
                for jl in range(_GB):
                    i = jl % _RING
                    g_wait(jl, i, pc)
                    s_start(jl, i, pc)
                    jn = jl + _DG
                    if jn < _GB:
                        ing = jn % _RING
                        if jn >= _RING:
                            s_wait(jn - _RING, ing, pc)
                        g_start(jn, ing, pc)
                for jl in range(_GB - _RING, _GB):
                    s_wait(jl, jl % _RING, pc)

                @pl.when(g + 2 < ng)
                def _():
                    stage_start(g + 2)

                @pl.when(g + 1 < ng)
                def _():
                    stage_wait(g + 1)
                    shift((g + 1) % 2, c)
                return carry

            lax.fori_loop(0, ng, group, 0)
            plsc.subcore_barrier()
            off = (cc * C + c) * NP + ss * RPS
            pltpu.sync_copy(acc.at[pl.ds(ss * RPS, RPS)],
                            p_hbm.at[pl.ds(off, RPS)])
    return body


@functools.lru_cache(maxsize=None)
def _spmm_kernel(C):
    return pl.kernel(
        _make_spmm_body(C),
        out_type=jax.ShapeDtypeStruct((NC * C * NP, CL), jnp.float32),
        mesh=_MESH,
        scratch_types=[
            pltpu.VMEM((2 * _PW,), jnp.int32),
            pltpu.VMEM((2, _GB, _SB), jnp.int32),
            pltpu.VMEM((_RING, _SB, CL), jnp.float32),
            pltpu.VMEM((_ZR, CL), jnp.float32),
            pltpu.VMEM_SHARED((NP, CL), jnp.float32),
            pltpu.SemaphoreType.DMA((_RING,)),
            pltpu.SemaphoreType.DMA((_RING,)),
            pltpu.SemaphoreType.DMA,
            pltpu.SemaphoreType.DMA((2,)),
        ],
        compiler_params=_SC_PARAMS,
    )


def _dis_body(degp_ref, dis_ref):
    deg = degp_ref[0, :, 0:1] + degp_ref[1, :, 0:1]
    pos = lax.broadcasted_iota(jnp.int32, (NP, 1), 0) < N
    ok = jnp.logical_and(pos, deg > 0)
    dis_ref[...] = jnp.where(ok, lax.rsqrt(jnp.maximum(deg, 1.0)), 0.0)


_dis_kernel = pl.pallas_call(
    _dis_body, out_shape=jax.ShapeDtypeStruct((NP, 1), jnp.float32))

_BR = 2048


def _scale_a_body(h_ref, dis_ref, g_ref):
    d = dis_ref[...]
    g_ref[0] = h_ref[:, :CL] * d
    g_ref[1] = h_ref[:, CL:] * d


def _scale_b_body(p0_ref, p1_ref, dis_ref, g2_ref, t1_ref):
    s = p0_ref[0] + p1_ref[0]
    d = dis_ref[...]
    g2_ref[...] = s * (d * d)
    t1_ref[...] = jnp.concatenate([s[0] * (-d), s[1] * (-d)], axis=1)


def _scale_c_body(p0_ref, p1_ref, dis_ref, t2_ref):
    s = p0_ref[0] + p1_ref[0]
    d = dis_ref[...]
    t2_ref[...] = jnp.concatenate([s[0] * (2.0 * d), s[1] * (2.0 * d)],
                                  axis=1)


@functools.lru_cache(maxsize=None)
def _scale_a(C):
    return pl.pallas_call(
        _scale_a_body,
        grid=(C // 2, NP // _BR),
        in_specs=[
            pl.BlockSpec((_BR, 2 * CL), lambda c, m: (m, c)),
            pl.BlockSpec((_BR, 1), lambda c, m: (m, 0)),
        ],
        out_specs=pl.BlockSpec((2, _BR, CL), lambda c, m: (c, m, 0)),
        out_shape=jax.ShapeDtypeStruct((C, NP, CL), jnp.float32),
    )


@functools.lru_cache(maxsize=None)
def _scale_b(C):
    return pl.pallas_call(
        _scale_b_body,
        grid=(C // 2, NP // _BR),
        in_specs=[
            pl.BlockSpec((1, 2, _BR, CL), lambda c, m: (0, c, m, 0)),
            pl.BlockSpec((1, 2, _BR, CL), lambda c, m: (1, c, m, 0)),
            pl.BlockSpec((_BR, 1), lambda c, m: (m, 0)),
        ],
        out_specs=[
            pl.BlockSpec((2, _BR, CL), lambda c, m: (c, m, 0)),
            pl.BlockSpec((_BR, 2 * CL), lambda c, m: (m, c)),
        ],
        out_shape=[
            jax.ShapeDtypeStruct((C, NP, CL), jnp.float32),
            jax.ShapeDtypeStruct((NP, C * CL), jnp.float32),
        ],
    )


@functools.lru_cache(maxsize=None)
def _scale_c(C):
    return pl.pallas_call(
        _scale_c_body,
        grid=(C // 2, NP // _BR),
        in_specs=[
            pl.BlockSpec((1, 2, _BR, CL), lambda c, m: (0, c, m, 0)),
            pl.BlockSpec((1, 2, _BR, CL), lambda c, m: (1, c, m, 0)),
            pl.BlockSpec((_BR, 1), lambda c, m: (m, 0)),
        ],
        out_specs=pl.BlockSpec((_BR, 2 * CL), lambda c, m: (m, c)),
        out_shape=jax.ShapeDtypeStruct((NP, C * CL), jnp.float32),
    )


def _mm_body(a_ref, w_ref, b_ref, o_ref, acc_ref, *, nk):
    k = pl.program_id(2)

    @pl.when(k == 0)
    def _():
        acc_ref[...] = jnp.zeros_like(acc_ref)

    acc_ref[...] += jnp.dot(a_ref[...], w_ref[...],
                            preferred_element_type=jnp.float32)

    @pl.when(k == nk - 1)
    def _():
        o_ref[...] = jnp.maximum(acc_ref[...] + b_ref[...], 0.0)


@functools.lru_cache(maxsize=None)
def _matmul(kdim, dpo):
    bm = 512
    bn = min(512, dpo)
    bk = 128
    nk = kdim // bk
    body = functools.partial(_mm_body, nk=nk)
    return pl.pallas_call(
        body,
        grid=(NP // bm, dpo // bn, nk),
        in_specs=[
            pl.BlockSpec((bm, bk), lambda m, n, k: (m, k)),
            pl.BlockSpec((bk, bn), lambda m, n, k: (k, n)),
            pl.BlockSpec((1, bn), lambda m, n, k: (0, n)),
        ],
        out_specs=pl.BlockSpec((bm, bn), lambda m, n, k: (m, n)),
        out_shape=jax.ShapeDtypeStruct((NP, dpo), jnp.float32),
        scratch_shapes=[pltpu.VMEM((bm, bn), jnp.float32)],
        compiler_params=pltpu.CompilerParams(
            dimension_semantics=("parallel", "parallel", "arbitrary")),
    )


_LAYERS = (
    (2, 128, 250, 256),
    (4, 250, 500, 512),
    (8, 500, 1000, 1024),
)


def kernel(x, edge_index, W1, b1, W2, b2, W3, b3):
    row = edge_index[0].astype(jnp.int32)
    col = edge_index[1].astype(jnp.int32)
    padn = EPAD - E
    rowp = jnp.concatenate([row, jnp.full((padn,), N, jnp.int32)])
    colp = jnp.concatenate([col, jnp.full((padn,), N, jnp.int32)])
    row3 = rowp.reshape(NW, NB, B)
    row2 = rowp.reshape(_TB, _SB)

    deg_parts = _deg_kernel(row3)
    dis = _dis_kernel(deg_parts.reshape(NC, NP, 16))

    H = jnp.pad(x, ((0, NP - N), (0, 0)))

    weights = ((W1, b1), (W2, b2), (W3, b3))
    for li, (C, din, dout, dpo) in enumerate(_LAYERS):
        W, b = weights[li]
        dp = C * CL
        Wp = jnp.pad(W, ((0, 0), (0, dp - din), (0, dpo - dout)))
        Wc = jnp.concatenate([Wp[0] - Wp[2], Wp[1], Wp[2]], axis=0)
        bc = jnp.pad(b, (0, dpo - dout)).reshape(1, dpo)

        G1 = _scale_a(C)(H, dis)
        P1 = _spmm_kernel(C)(G1.reshape(C * NP, CL), colp, row2)
        G2, T1 = _scale_b(C)(P1.reshape(NC, C, NP, CL),
                             P1.reshape(NC, C, NP, CL), dis)
        P2 = _spmm_kernel(C)(G2.reshape(C * NP, CL), colp, row2)
        T2 = _scale_c(C)(P2.reshape(NC, C, NP, CL),
                         P2.reshape(NC, C, NP, CL), dis)

        A = jnp.concatenate([H, T1, T2], axis=1)
        out = _matmul(3 * dp, dpo)(A, Wc, bc)
        if li == len(_LAYERS) - 1:
            return out[:N, :1000]
        H = out


# speedup vs baseline: 1.0105x; 1.0105x over previous
import functools

import jax
import jax.numpy as jnp
from jax import lax
from jax.experimental import pallas as pl
from jax.experimental.pallas import tpu as pltpu
from jax.experimental.pallas import tpu_sc as plsc

N = 10000
NP = 10240
E = 320000
NC, NS = 2, 16
NW = NC * NS
B = 128
NB = 80
EW = NB * B
EPAD = NW * EW
RPS = NP // NS
CL = 64

_MESH = plsc.VectorSubcoreMesh(core_axis_name="c", subcore_axis_name="s")
_SC_PARAMS = pltpu.CompilerParams(use_tc_tiling_on_sc=False)


def _deg_body(row_hbm, deg_hbm, row_v, ones_v, zb, acc):
    cc = lax.axis_index("c")
    ss = lax.axis_index("s")
    wid = ss * NC + cc
    pltpu.sync_copy(row_hbm.at[wid], row_v)
    one = jnp.ones((16,), jnp.float32)
    zv = jnp.zeros((16,), jnp.float32)
    for r in range(B):
        ones_v[r, 0:16] = one
    for r in range(128):
        zb[r, 0:16] = zv

    def zstripe(t, carry):
        pltpu.sync_copy(zb, acc.at[pl.ds(ss * RPS + t * 128, 128)])
        return carry

    lax.fori_loop(0, RPS // 128, zstripe, 0)
    plsc.subcore_barrier()

    def batch(j, carry):
        pltpu.sync_copy(ones_v, acc.at[row_v.at[j]], add=True)
        return carry

    lax.fori_loop(0, NB, batch, 0)
    plsc.subcore_barrier()
    pltpu.sync_copy(acc.at[pl.ds(ss * RPS, RPS)],
                    deg_hbm.at[pl.ds(cc * NP + ss * RPS, RPS)])


_deg_kernel = pl.kernel(
    _deg_body,
    out_type=jax.ShapeDtypeStruct((NC * NP, 16), jnp.float32),
    mesh=_MESH,
    scratch_types=[
        pltpu.VMEM((NB, B), jnp.int32),
        pltpu.VMEM((B, 16), jnp.float32),
        pltpu.VMEM((128, 16), jnp.float32),
        pltpu.VMEM_SHARED((NP, 16), jnp.float32),
    ],
    compiler_params=_SC_PARAMS,
)


_RING = 8
_DG = 6
_ZR = 64
_GB = 32

_FAST = 0
_SB = 128
_NBF = 128
_NBS = 32
_TB = EPAD // _SB
_PB = _TB // NS
_PW = _GB * _SB


def _make_spmm_body(C):
    def body(g_hbm, colf_hbm, row2_hbm, p_hbm,
             col_v, row_v, gbufs, zb, acc, gsems, ssems, zsem, tsems):
        cc = lax.axis_index("c")
        ss = lax.axis_index("s")
        fast = cc == _FAST
        ng = jnp.where(fast, _NBF // _GB, _NBS // _GB)
        base = ss * _PB + jnp.where(fast, 0, _NBF)

        def stage_start(g):
            pc = g % 2
            pltpu.async_copy(colf_hbm.at[pl.ds((base + g * _GB) * _SB, _PW)],
                             col_v.at[pl.ds(pc * _PW, _PW)], tsems.at[pc])
            pltpu.async_copy(row2_hbm.at[pl.ds(base + g * _GB, _GB)],
                             row_v.at[pc], tsems.at[pc])

        def stage_wait(g):
            pc = g % 2
            pltpu.make_async_copy(
                colf_hbm.at[pl.ds((base + g * _GB) * _SB, _PW)],
                col_v.at[pl.ds(pc * _PW, _PW)], tsems.at[pc]).wait()
            pltpu.make_async_copy(row2_hbm.at[pl.ds(base + g * _GB, _GB)],
                                  row_v.at[pc], tsems.at[pc]).wait()

        def shift(pc, c):
            def go(t, carry):
                sl = pl.ds(pc * _PW + t * 16, 16)
                col_v[sl] = col_v[sl] + c * NP
                return carry
            lax.fori_loop(0, _PW // 16, go, 0)

        zv = jnp.zeros((16,), jnp.float32)
        for r in range(_ZR):
            for k in range(CL // 16):
                zb[r, k * 16:(k + 1) * 16] = zv

        def g_start(jl, i, pc):
            pltpu.async_copy(
                g_hbm.at[col_v.at[pl.ds(pc * _PW + jl * _SB, _SB)]],
                gbufs.at[i], gsems.at[i])

        def g_wait(jl, i, pc):
            pltpu.make_async_copy(
                g_hbm.at[col_v.at[pl.ds(pc * _PW + jl * _SB, _SB)]],
                gbufs.at[i], gsems.at[i]).wait()

        def s_start(jl, i, pc):
            pltpu.async_copy(gbufs.at[i], acc.at[row_v.at[pc].at[jl]],
                             ssems.at[i], add=True)

        def s_wait(jl, i, pc):
            pltpu.make_async_copy(gbufs.at[i], acc.at[row_v.at[pc].at[jl]],
                                  ssems.at[i]).wait()

        for c in range(C):
            for t in range(RPS // _ZR):
                pltpu.async_copy(zb, acc.at[pl.ds(ss * RPS + t * _ZR, _ZR)],
                                 zsem)
            for t in range(RPS // _ZR):
                pltpu.make_async_copy(
                    zb, acc.at[pl.ds(ss * RPS + t * _ZR, _ZR)], zsem).wait()
            plsc.subcore_barrier()

            stage_start(0)
            stage_wait(0)
            shift(0, c)
            for i in range(_DG):
                g_start(i, i, 0)

            def group(g, carry):
                cur = g % 2
                nxt = (g + 1) % 2
                has_next = g + 1 < ng

                for jl in range(_GB):
                    i = jl % _RING
                    g_wait(jl, i, cur)
                    s_start(jl, i, cur)
                    if jl < _GB - _DG:
                        ing = (jl + _DG) % _RING
                        if jl >= _RING - _DG:
                            s_wait(jl - (_RING - _DG), ing, cur)
                        g_start(jl + _DG, ing, cur)
                    else:
                        @pl.when(has_next)
                        def _():
                            jn = jl + _DG - _GB
                            ing = jn % _RING
                            s_wait(jl - (_RING - _DG), ing, cur)
                            g_start(jn, ing, nxt)
                    if jl == 4:
                        @pl.when(has_next)
                        def _():
                            stage_start(g + 1)
                    if jl == 20:
                        @pl.when(has_next)
                        def _():
                            stage_wait(g + 1)
                            shift(nxt, c)

                @pl.when(has_next)
                def _():
                    for jl in range(_GB - (_RING - _DG), _GB):
                        s_wait(jl, jl % _RING, cur)
                return carry

            lax.fori_loop(0, ng, group, 0)
            pl_last = (ng - 1) % 2
            for jl in range(_GB - _RING, _GB):
                s_wait(jl, jl % _RING, pl_last)
            plsc.subcore_barrier()
            off = (cc * C + c) * NP + ss * RPS
            pltpu.sync_copy(acc.at[pl.ds(ss * RPS, RPS)],
                            p_hbm.at[pl.ds(off, RPS)])
    return body


@functools.lru_cache(maxsize=None)
def _spmm_kernel(C):
    return pl.kernel(
        _make_spmm_body(C),
        out_type=jax.ShapeDtypeStruct((NC * C * NP, CL), jnp.float32),
        mesh=_MESH,
        scratch_types=[
            pltpu.VMEM((2 * _PW,), jnp.int32),
            pltpu.VMEM((2, _GB, _SB), jnp.int32),
            pltpu.VMEM((_RING, _SB, CL), jnp.float32),
            pltpu.VMEM((_ZR, CL), jnp.float32),
            pltpu.VMEM_SHARED((NP, CL), jnp.float32),
            pltpu.SemaphoreType.DMA((_RING,)),
            pltpu.SemaphoreType.DMA((_RING,)),
            pltpu.SemaphoreType.DMA,
            pltpu.SemaphoreType.DMA((2,)),
        ],
        compiler_params=_SC_PARAMS,
    )


def _dis_body(degp_ref, dis_ref):
    deg = degp_ref[0, :, 0:1] + degp_ref[1, :, 0:1]
    pos = lax.broadcasted_iota(jnp.int32, (NP, 1), 0) < N
    ok = jnp.logical_and(pos, deg > 0)
    dis_ref[...] = jnp.where(ok, lax.rsqrt(jnp.maximum(deg, 1.0)), 0.0)


_dis_kernel = pl.pallas_call(
    _dis_body, out_shape=jax.ShapeDtypeStruct((NP, 1), jnp.float32))

_BR = 2048


def _scale_a_body(h_ref, dis_ref, g_ref):
    d = dis_ref[...]
    g_ref[0] = h_ref[:, :CL] * d
    g_ref[1] = h_ref[:, CL:] * d


def _scale_b_body(p0_ref, p1_ref, dis_ref, g2_ref, t1_ref):
    s = p0_ref[0] + p1_ref[0]
    d = dis_ref[...]
    g2_ref[...] = s * (d * d)
    t1_ref[...] = jnp.concatenate([s[0] * (-d), s[1] * (-d)], axis=1)


def _scale_c_body(p0_ref, p1_ref, dis_ref, t2_ref):
    s = p0_ref[0] + p1_ref[0]
    d = dis_ref[...]
    t2_ref[...] = jnp.concatenate([s[0] * (2.0 * d), s[1] * (2.0 * d)],
                                  axis=1)


@functools.lru_cache(maxsize=None)
def _scale_a(C):
    return pl.pallas_call(
        _scale_a_body,
        grid=(C // 2, NP // _BR),
        in_specs=[
            pl.BlockSpec((_BR, 2 * CL), lambda c, m: (m, c)),
            pl.BlockSpec((_BR, 1), lambda c, m: (m, 0)),
        ],
        out_specs=pl.BlockSpec((2, _BR, CL), lambda c, m: (c, m, 0)),
        out_shape=jax.ShapeDtypeStruct((C, NP, CL), jnp.float32),
    )


@functools.lru_cache(maxsize=None)
def _scale_b(C):
    return pl.pallas_call(
        _scale_b_body,
        grid=(C // 2, NP // _BR),
        in_specs=[
            pl.BlockSpec((1, 2, _BR, CL), lambda c, m: (0, c, m, 0)),
            pl.BlockSpec((1, 2, _BR, CL), lambda c, m: (1, c, m, 0)),
            pl.BlockSpec((_BR, 1), lambda c, m: (m, 0)),
        ],
        out_specs=[
            pl.BlockSpec((2, _BR, CL), lambda c, m: (c, m, 0)),
            pl.BlockSpec((_BR, 2 * CL), lambda c, m: (m, c)),
        ],
        out_shape=[
            jax.ShapeDtypeStruct((C, NP, CL), jnp.float32),
            jax.ShapeDtypeStruct((NP, C * CL), jnp.float32),
        ],
    )


@functools.lru_cache(maxsize=None)
def _scale_c(C):
    return pl.pallas_call(
        _scale_c_body,
        grid=(C // 2, NP // _BR),
        in_specs=[
            pl.BlockSpec((1, 2, _BR, CL), lambda c, m: (0, c, m, 0)),
            pl.BlockSpec((1, 2, _BR, CL), lambda c, m: (1, c, m, 0)),
            pl.BlockSpec((_BR, 1), lambda c, m: (m, 0)),
        ],
        out_specs=pl.BlockSpec((_BR, 2 * CL), lambda c, m: (m, c)),
        out_shape=jax.ShapeDtypeStruct((NP, C * CL), jnp.float32),
    )


def _mm_body(a_ref, w_ref, b_ref, o_ref, acc_ref, *, nk):
    k = pl.program_id(2)

    @pl.when(k == 0)
    def _():
        acc_ref[...] = jnp.zeros_like(acc_ref)

    acc_ref[...] += jnp.dot(a_ref[...], w_ref[...],
                            preferred_element_type=jnp.float32)

    @pl.when(k == nk - 1)
    def _():
        o_ref[...] = jnp.maximum(acc_ref[...] + b_ref[...], 0.0)


@functools.lru_cache(maxsize=None)
def _matmul(kdim, dpo):
    bm = 512
    bn = min(512, dpo)
    bk = 128
    nk = kdim // bk
    body = functools.partial(_mm_body, nk=nk)
    return pl.pallas_call(
        body,
        grid=(NP // bm, dpo // bn, nk),
        in_specs=[
            pl.BlockSpec((bm, bk), lambda m, n, k: (m, k)),
            pl.BlockSpec((bk, bn), lambda m, n, k: (k, n)),
            pl.BlockSpec((1, bn), lambda m, n, k: (0, n)),
        ],
        out_specs=pl.BlockSpec((bm, bn), lambda m, n, k: (m, n)),
        out_shape=jax.ShapeDtypeStruct((NP, dpo), jnp.float32),
        scratch_shapes=[pltpu.VMEM((bm, bn), jnp.float32)],
        compiler_params=pltpu.CompilerParams(
            dimension_semantics=("parallel", "parallel", "arbitrary")),
    )


_LAYERS = (
    (2, 128, 250, 256),
    (4, 250, 500, 512),
    (8, 500, 1000, 1024),
)


def kernel(x, edge_index, W1, b1, W2, b2, W3, b3):
    row = edge_index[0].astype(jnp.int32)
    col = edge_index[1].astype(jnp.int32)
    padn = EPAD - E
    rowp = jnp.concatenate([row, jnp.full((padn,), N, jnp.int32)])
    colp = jnp.concatenate([col, jnp.full((padn,), N, jnp.int32)])
    row3 = rowp.reshape(NW, NB, B)
    row2 = rowp.reshape(_TB, _SB)

    deg_parts = _deg_kernel(row3)
    dis = _dis_kernel(deg_parts.reshape(NC, NP, 16))

    H = jnp.pad(x, ((0, NP - N), (0, 0)))

    weights = ((W1, b1), (W2, b2), (W3, b3))
    for li, (C, din, dout, dpo) in enumerate(_LAYERS):
        W, b = weights[li]
        dp = C * CL
        Wp = jnp.pad(W, ((0, 0), (0, dp - din), (0, dpo - dout)))
        Wc = jnp.concatenate([Wp[0] - Wp[2], Wp[1], Wp[2]], axis=0)
        bc = jnp.pad(b, (0, dpo - dout)).reshape(1, dpo)

        G1 = _scale_a(C)(H, dis)
        P1 = _spmm_kernel(C)(G1.reshape(C * NP, CL), colp, row2)
        G2, T1 = _scale_b(C)(P1.reshape(NC, C, NP, CL),
                             P1.reshape(NC, C, NP, CL), dis)
        P2 = _spmm_kernel(C)(G2.reshape(C * NP, CL), colp, row2)
        T2 = _scale_c(C)(P2.reshape(NC, C, NP, CL),
                         P2.reshape(NC, C, NP, CL), dis)

        A = jnp.concatenate([H, T1, T2], axis=1)
        out = _matmul(3 * dp, dpo)(A, Wc, bc)
        if li == len(_LAYERS) - 1:
            return out[:N, :1000]
        H = out
